# baseline (speedup 1.0000x reference)
import functools

import jax
import jax.numpy as jnp
from jax import lax
from jax.experimental import pallas as pl
from jax.experimental.pallas import tpu as pltpu
from jax.experimental.pallas import tpu_sc as plsc

_VOCAB = 100000
_DIM = 64
_B, _F = 4096, 26
_N = _B * _F
_NW = 32
_BPW = _B // _NW


def _body(idx_hbm, table_hbm, out_hbm, idx_v, idx_t, rows_a, rows_b,
          stage_a, stage_b, sem_a, sem_b, sem_oa, sem_ob):
    wid = lax.axis_index("s") * 2 + lax.axis_index("c")
    base = wid * (_BPW * _F)
    pltpu.sync_copy(idx_hbm.at[pl.ds(base, _BPW * _F)], idx_v)

    iota = lax.iota(jnp.int32, 16)
    iota26 = iota * _F
    zeros = jnp.zeros((16,), jnp.int32)
    for f in range(_F):
        for b in range(8):
            src = iota26 + (b * 16 * _F + f)
            idx_t[f, pl.ds(b * 16, 16)] = plsc.load_gather(idx_v, [src])
    for f in (_F, _F + 1):
        for b in range(8):
            idx_t[f, pl.ds(b * 16, 16)] = zeros

    def fire_gather(f, rows_v, sem_g):
        pltpu.async_copy(table_hbm.at[idx_t.at[f]], rows_v, sem_g)

    def drain(src, dst, sem):
        pltpu.make_async_copy(src, dst, sem).wait()

    def do_feature(f, rows_v, sem_g, stage, sem_o):
        drain(table_hbm.at[idx_t.at[f]], rows_v, sem_g)
        for g in range(8):
            for r in range(8):
                col = jnp.full((16,), 8 * g + r, jnp.int32)
                for b in range(8):
                    v = plsc.load_gather(rows_v, [iota + b * 16, col])
                    stage[g, r, pl.ds(b * 16, 16)] = v
        for g in range(8):
            pltpu.async_copy(stage.at[g], out_hbm.at[f, g, wid], sem_o)
        fire_gather(f + 2, rows_v, sem_g)
        for g in range(8):
            drain(stage.at[g], out_hbm.at[f, g, wid], sem_o)

    fire_gather(0, rows_a, sem_a)
    fire_gather(1, rows_b, sem_b)

    def loop_body(k, carry):
        do_feature(2 * k, rows_a, sem_a, stage_a, sem_oa)
        do_feature(2 * k + 1, rows_b, sem_b, stage_b, sem_ob)
        return carry

    lax.fori_loop(0, _F // 2, loop_body, 0)
    drain(table_hbm.at[idx_t.at[_F]], rows_a, sem_a)
    drain(table_hbm.at[idx_t.at[_F + 1]], rows_b, sem_b)


_gather = pl.kernel(
    _body,
    mesh=plsc.VectorSubcoreMesh(core_axis_name="c", subcore_axis_name="s"),
    compiler_params=pltpu.CompilerParams(
        use_tc_tiling_on_sc=False, needs_layout_passes=False
    ),
    out_type=jax.ShapeDtypeStruct((_F, 8, _NW, 8, 128), jnp.float32),
    scratch_types=[
        pltpu.VMEM((_BPW * _F,), jnp.int32),
        pltpu.VMEM((_F + 2, 128), jnp.int32),
        pltpu.VMEM((128, _DIM), jnp.float32),
        pltpu.VMEM((128, _DIM), jnp.float32),
        pltpu.VMEM((8, 8, 128), jnp.float32),
        pltpu.VMEM((8, 8, 128), jnp.float32),
        pltpu.SemaphoreType.DMA,
        pltpu.SemaphoreType.DMA,
        pltpu.SemaphoreType.DMA,
        pltpu.SemaphoreType.DMA,
    ],
)


@jax.jit
def kernel(ids, table):
    t5 = _gather(ids.reshape(_N), table)
    out = t5.transpose(2, 4, 0, 1, 3).reshape(_B, _F, _DIM)
    return out


# speedup vs baseline: 1.1409x; 1.1409x over previous
import functools

import jax
import jax.numpy as jnp
from jax import lax
from jax.experimental import pallas as pl
from jax.experimental.pallas import tpu as pltpu
from jax.experimental.pallas import tpu_sc as plsc

_VOCAB = 100000
_DIM = 64
_B, _F = 4096, 26
_N = _B * _F
_NW = 32
_BPW = _B // _NW


def _body(idx_hbm, table_hbm, out_hbm, idx_v, idx_t, rows_a, rows_b,
          stage_a, stage_b, sem_a, sem_b, sem_oa, sem_ob):
    wid = lax.axis_index("s") * 2 + lax.axis_index("c")
    base = wid * (_BPW * _F)
    pltpu.sync_copy(idx_hbm.at[pl.ds(base, _BPW * _F)], idx_v)

    iota = lax.iota(jnp.int32, 16)
    iota26 = iota * _F
    zeros = jnp.zeros((16,), jnp.int32)
    for f in range(_F):
        for b in range(8):
            src = iota26 + (b * 16 * _F + f)
            idx_t[f, pl.ds(b * 16, 16)] = plsc.load_gather(idx_v, [src])
    for f in (_F, _F + 1):
        for b in range(8):
            idx_t[f, pl.ds(b * 16, 16)] = zeros

    def fire_gather(f, rows_v, sem_g):
        pltpu.async_copy(table_hbm.at[idx_t.at[f]], rows_v, sem_g)

    def drain(src, dst, sem):
        pltpu.make_async_copy(src, dst, sem).wait()

    def do_feature(f, rows_v, sem_g, stage, sem_o):
        drain(table_hbm.at[idx_t.at[f]], rows_v, sem_g)
        for g in range(8):
            for r in range(8):
                col = jnp.full((16,), 8 * g + r, jnp.int32)
                vs = [
                    plsc.load_gather(rows_v, [iota + b * 16, col])
                    for b in range(8)
                ]
                for b in range(8):
                    stage[g, r, pl.ds(b * 16, 16)] = vs[b]
        copies = [
            pltpu.async_copy(stage.at[g], out_hbm.at[f, g, wid], sem_o)
            for g in range(8)
        ]
        fire_gather(f + 2, rows_v, sem_g)
        for c in copies:
            c.wait()

    fire_gather(0, rows_a, sem_a)
    fire_gather(1, rows_b, sem_b)

    def loop_body(k, carry):
        do_feature(2 * k, rows_a, sem_a, stage_a, sem_oa)
        do_feature(2 * k + 1, rows_b, sem_b, stage_b, sem_ob)
        return carry

    lax.fori_loop(0, _F // 2, loop_body, 0)
    drain(table_hbm.at[idx_t.at[_F]], rows_a, sem_a)
    drain(table_hbm.at[idx_t.at[_F + 1]], rows_b, sem_b)


_gather = pl.kernel(
    _body,
    mesh=plsc.VectorSubcoreMesh(core_axis_name="c", subcore_axis_name="s"),
    compiler_params=pltpu.CompilerParams(
        use_tc_tiling_on_sc=False, needs_layout_passes=False
    ),
    out_type=jax.ShapeDtypeStruct((_F, 8, _NW, 8, 128), jnp.float32),
    scratch_types=[
        pltpu.VMEM((_BPW * _F,), jnp.int32),
        pltpu.VMEM((_F + 2, 128), jnp.int32),
        pltpu.VMEM((128, _DIM), jnp.float32),
        pltpu.VMEM((128, _DIM), jnp.float32),
        pltpu.VMEM((8, 8, 128), jnp.float32),
        pltpu.VMEM((8, 8, 128), jnp.float32),
        pltpu.SemaphoreType.DMA,
        pltpu.SemaphoreType.DMA,
        pltpu.SemaphoreType.DMA,
        pltpu.SemaphoreType.DMA,
    ],
)


@jax.jit
def kernel(ids, table):
    t5 = _gather(ids.reshape(_N), table)
    out = t5.transpose(2, 4, 0, 1, 3).reshape(_B, _F, _DIM)
    return out


# speedup vs baseline: 1.7116x; 1.5002x over previous
import functools

import jax
import jax.numpy as jnp
from jax import lax
from jax.experimental import pallas as pl
from jax.experimental.pallas import tpu as pltpu
from jax.experimental.pallas import tpu_sc as plsc

_VOCAB = 100000
_DIM = 64
_B, _F = 4096, 26
_N = _B * _F
_NW = 32
_BPW = _B // _NW


def _body(idx_hbm, table_hbm, out_hbm, idx_v, idx_t, rows_a, rows_b,
          stage_a, stage_b, sem_a, sem_b, sem_oa, sem_ob):
    wid = lax.axis_index("s") * 2 + lax.axis_index("c")
    base = wid * (_BPW * _F)
    pltpu.sync_copy(idx_hbm.at[pl.ds(base, _BPW * _F)], idx_v)

    iota = lax.iota(jnp.int32, 16)
    iota26 = iota * _F
    zeros = jnp.zeros((16,), jnp.int32)
    for f in range(_F):
        for b in range(8):
            src = iota26 + (b * 16 * _F + f)
            idx_t[f, pl.ds(b * 16, 16)] = plsc.load_gather(idx_v, [src])
    for f in (_F, _F + 1):
        for b in range(8):
            idx_t[f, pl.ds(b * 16, 16)] = zeros

    def fire_gather(f, rows_v, sem_g):
        pltpu.async_copy(table_hbm.at[idx_t.at[f]], rows_v, sem_g)

    def drain(src, dst, sem):
        pltpu.make_async_copy(src, dst, sem).wait()

    def do_feature(f, rows_v, sem_g, stage, sem_o):
        pltpu.async_copy(table_hbm.at[idx_t.at[f]], rows_v, sem_g).wait()
        for g in range(8):
            for r in range(8):
                col = jnp.full((16,), 8 * g + r, jnp.int32)
                vs = [
                    plsc.load_gather(rows_v, [iota + b * 16, col])
                    for b in range(8)
                ]
                for b in range(8):
                    stage[g, r, pl.ds(b * 16, 16)] = vs[b]
        copies = [
            pltpu.async_copy(stage.at[g], out_hbm.at[f, g, wid], sem_o)
            for g in range(8)
        ]
        for c in copies:
            c.wait()

    def loop_body(k, carry):
        do_feature(2 * k, rows_a, sem_a, stage_a, sem_oa)
        do_feature(2 * k + 1, rows_b, sem_b, stage_b, sem_ob)
        return carry

    lax.fori_loop(0, _F // 2, loop_body, 0)


_gather = pl.kernel(
    _body,
    mesh=plsc.VectorSubcoreMesh(core_axis_name="c", subcore_axis_name="s"),
    compiler_params=pltpu.CompilerParams(
        use_tc_tiling_on_sc=False, needs_layout_passes=False
    ),
    out_type=jax.ShapeDtypeStruct((_F, 8, _NW, 8, 128), jnp.float32),
    scratch_types=[
        pltpu.VMEM((_BPW * _F,), jnp.int32),
        pltpu.VMEM((_F + 2, 128), jnp.int32),
        pltpu.VMEM((128, _DIM), jnp.float32),
        pltpu.VMEM((128, _DIM), jnp.float32),
        pltpu.VMEM((8, 8, 128), jnp.float32),
        pltpu.VMEM((8, 8, 128), jnp.float32),
        pltpu.SemaphoreType.DMA,
        pltpu.SemaphoreType.DMA,
        pltpu.SemaphoreType.DMA,
        pltpu.SemaphoreType.DMA,
    ],
)


@jax.jit
def kernel(ids, table):
    t5 = _gather(ids.reshape(_N), table)
    out = t5.transpose(2, 4, 0, 1, 3).reshape(_B, _F, _DIM)
    return out


# speedup vs baseline: 1.8598x; 1.0866x over previous
import functools

import jax
import jax.numpy as jnp
from jax import lax
from jax.experimental import pallas as pl
from jax.experimental.pallas import tpu as pltpu
from jax.experimental.pallas import tpu_sc as plsc

_VOCAB = 100000
_DIM = 64
_B, _F = 4096, 26
_N = _B * _F
_NW = 32
_BPW = _B // _NW


_CHUNKS = (5, 5, 4, 4, 4, 4)
_OFFS = (0, 5, 10, 14, 18, 22)


def _body(idx_hbm, table_hbm, out_hbm, idx_v, idx_ft, buf_a, buf_b,
          stage, sem_a, sem_b, sem_o):
    wid = lax.axis_index("s") * 2 + lax.axis_index("c")
    base = wid * (_BPW * _F)
    pltpu.sync_copy(idx_hbm.at[pl.ds(base, _BPW * _F)], idx_v)

    iota = lax.iota(jnp.int32, 16)
    iota26 = iota * _F
    for f in range(_F):
        vs = [
            plsc.load_gather(idx_v, [iota26 + (b * 16 * _F + f)])
            for b in range(8)
        ]
        for b in range(8):
            idx_ft[pl.ds(f * 128 + b * 16, 16)] = vs[b]

    def fire(c, buf, sem):
        n = _CHUNKS[c] * 128
        return pltpu.async_copy(
            table_hbm.at[idx_ft.at[pl.ds(_OFFS[c] * 128, n)]],
            buf.at[pl.ds(0, n)],
            sem,
        )

    def feat(f, rows):
        for g in range(8):
            for r in range(8):
                col = jnp.full((16,), 8 * g + r, jnp.int32)
                vs = [
                    plsc.load_gather(rows, [iota + b * 16, col])
                    for b in range(8)
                ]
                for b in range(8):
                    stage[g, r, pl.ds(b * 16, 16)] = vs[b]
        copies = [
            pltpu.async_copy(stage.at[g], out_hbm.at[f, g, wid], sem_o)
            for g in range(8)
        ]
        for c in copies:
            c.wait()

    bufs = (buf_a, buf_b)
    sems = (sem_a, sem_b)
    cp = fire(0, buf_a, sem_a)
    for c in range(len(_CHUNKS)):
        cur = bufs[c % 2]
        cp.wait()
        if c + 1 < len(_CHUNKS):
            cp = fire(c + 1, bufs[(c + 1) % 2], sems[(c + 1) % 2])

        def feat_body(i, carry, c=c, cur=cur):
            feat(_OFFS[c] + i, cur.at[pl.ds(i * 128, 128)])
            return carry

        lax.fori_loop(0, _CHUNKS[c], feat_body, 0)


_gather = pl.kernel(
    _body,
    mesh=plsc.VectorSubcoreMesh(core_axis_name="c", subcore_axis_name="s"),
    compiler_params=pltpu.CompilerParams(
        use_tc_tiling_on_sc=False, needs_layout_passes=False
    ),
    out_type=jax.ShapeDtypeStruct((_F, 8, _NW, 8, 128), jnp.float32),
    scratch_types=[
        pltpu.VMEM((_BPW * _F,), jnp.int32),
        pltpu.VMEM((_BPW * _F,), jnp.int32),
        pltpu.VMEM((5 * 128, _DIM), jnp.float32),
        pltpu.VMEM((5 * 128, _DIM), jnp.float32),
        pltpu.VMEM((8, 8, 128), jnp.float32),
        pltpu.SemaphoreType.DMA,
        pltpu.SemaphoreType.DMA,
        pltpu.SemaphoreType.DMA,
    ],
)


@jax.jit
def kernel(ids, table):
    t5 = _gather(ids.reshape(_N), table)
    out = t5.transpose(2, 4, 0, 1, 3).reshape(_B, _F, _DIM)
    return out


# speedup vs baseline: 3.7065x; 1.9930x over previous
import functools

import jax
import jax.numpy as jnp
from jax import lax
from jax.experimental import pallas as pl
from jax.experimental.pallas import tpu as pltpu
from jax.experimental.pallas import tpu_sc as plsc

_VOCAB = 100000
_DIM = 64
_B, _F = 4096, 26
_N = _B * _F
_NW = 32
_BPW = _B // _NW


_CHUNKS = (5, 5, 4, 4, 4, 4)
_OFFS = (0, 5, 10, 14, 18, 22)


def _body(idx_hbm, table_hbm, out_hbm, idx_v, idx_ft, buf_a, buf_b,
          stage, sem_a, sem_b, sem_o):
    wid = lax.axis_index("s") * 2 + lax.axis_index("c")
    base = wid * (_BPW * _F)
    pltpu.sync_copy(idx_hbm.at[pl.ds(base, _BPW * _F)], idx_v)

    iota = lax.iota(jnp.int32, 16)
    iota26 = iota * _F
    for f in range(_F):
        vs = [
            plsc.load_gather(idx_v, [iota26 + (b * 16 * _F + f)])
            for b in range(8)
        ]
        for b in range(8):
            idx_ft[pl.ds(f * 128 + b * 16, 16)] = vs[b]

    def fire(c, buf, sem):
        n = _CHUNKS[c] * 128
        return pltpu.async_copy(
            table_hbm.at[idx_ft.at[pl.ds(_OFFS[c] * 128, n)]],
            buf.at[pl.ds(0, n)],
            sem,
        )

    def feat(f, rows):
        def jloop(t, carry):
            vms = [(iota + (4 * t + d)) & 63 for d in range(4)]
            for cb in range(8):
                cvec = iota + cb * 16
                vs = [plsc.load_gather(rows, [cvec, vm]) for vm in vms]
                for d in range(4):
                    plsc.store_scatter(stage, [vms[d], cvec], vs[d])
            return carry

        lax.fori_loop(0, 16, jloop, 0)
        copies = [
            pltpu.async_copy(
                stage.at[pl.ds(g * 8, 8)], out_hbm.at[f, g, wid], sem_o
            )
            for g in range(8)
        ]
        for c in copies:
            c.wait()

    bufs = (buf_a, buf_b)
    sems = (sem_a, sem_b)
    cp = fire(0, buf_a, sem_a)
    for c in range(len(_CHUNKS)):
        cur = bufs[c % 2]
        cp.wait()
        if c + 1 < len(_CHUNKS):
            cp = fire(c + 1, bufs[(c + 1) % 2], sems[(c + 1) % 2])

        def feat_body(i, carry, c=c, cur=cur):
            feat(_OFFS[c] + i, cur.at[pl.ds(i * 128, 128)])
            return carry

        lax.fori_loop(0, _CHUNKS[c], feat_body, 0)


_gather = pl.kernel(
    _body,
    mesh=plsc.VectorSubcoreMesh(core_axis_name="c", subcore_axis_name="s"),
    compiler_params=pltpu.CompilerParams(
        use_tc_tiling_on_sc=False, needs_layout_passes=False
    ),
    out_type=jax.ShapeDtypeStruct((_F, 8, _NW, 8, 128), jnp.float32),
    scratch_types=[
        pltpu.VMEM((_BPW * _F,), jnp.int32),
        pltpu.VMEM((_BPW * _F,), jnp.int32),
        pltpu.VMEM((5 * 128, _DIM), jnp.float32),
        pltpu.VMEM((5 * 128, _DIM), jnp.float32),
        pltpu.VMEM((64, 128), jnp.float32),
        pltpu.SemaphoreType.DMA,
        pltpu.SemaphoreType.DMA,
        pltpu.SemaphoreType.DMA,
    ],
)


@jax.jit
def kernel(ids, table):
    t5 = _gather(ids.reshape(_N), table)
    out = t5.transpose(2, 4, 0, 1, 3).reshape(_B, _F, _DIM)
    return out
